# Initial kernel scaffold; baseline (speedup 1.0000x reference)
import jax
import jax.numpy as jnp
from jax.experimental import pallas as pl


def kernel(z, user_ids, y):
    raise NotImplementedError("write your pallas kernel here")


# speedup vs baseline: 6.0953x; 6.0953x over previous
import functools

import jax
import jax.numpy as jnp
from jax import lax
from jax.experimental import pallas as pl
from jax.experimental.pallas import tpu as pltpu
from jax.experimental.pallas import tpu_sc as plsc

N = 320000
D = 128
NU = 10000
NC = 2
NS = 16
NW = NC * NS
LANES = 16

PAD_U = 10240
ZROWS = 80
CHUNK = 80
RPT = N // NW
CBC = PAD_U // NW
HCB = CBC // 2
YROWS = PAD_U // 8

_f32 = jnp.float32
_i32 = jnp.int32


def _mesh():
    return plsc.VectorSubcoreMesh(core_axis_name="c", subcore_axis_name="s")


def _stage_a(z, ids, y):
    @functools.partial(
        pl.kernel,
        out_type=(
            jax.ShapeDtypeStruct((NC, PAD_U, D), _f32),
            jax.ShapeDtypeStruct((NC, YROWS, D), _f32),
            jax.ShapeDtypeStruct((NC, PAD_U), _f32),
        ),
        mesh=_mesh(),
        scratch_types=[
            pltpu.VMEM((CHUNK, D), _f32),
            pltpu.VMEM((CHUNK,), _i32),
            pltpu.VMEM((CHUNK,), _i32),
            pltpu.VMEM((CHUNK,), _f32),
            pltpu.VMEM((CHUNK, D), _f32),
            pltpu.VMEM((CHUNK * 8,), _f32),
            pltpu.VMEM_SHARED((PAD_U, D), _f32),
            pltpu.VMEM_SHARED((YROWS, D), _f32),
        ],
    )
    def k(z_hbm, ids_hbm, y_hbm, zout_hbm, ycout_hbm, cnt_hbm,
          zbuf, idbuf, idx2buf, ybuf, ycbuf, cntbuf, zacc, ycacc):
        cid = lax.axis_index("c")
        sid = lax.axis_index("s")
        iota = lax.iota(_i32, LANES)
        zerov = jnp.zeros((LANES,), _f32)
        e0 = jnp.where(iota == 0, 1.0, 0.0).astype(_f32)
        e1 = jnp.where(iota == 1, 1.0, 0.0).astype(_f32)

        def zrow(r, _):
            for q in range(D // LANES):
                zbuf[r, pl.ds(q * LANES, LANES)] = zerov
                ycbuf[r, pl.ds(q * LANES, LANES)] = zerov
            return 0

        lax.fori_loop(0, ZROWS, zrow, 0)

        accbase = sid * (PAD_U // NS)
        for t in range(PAD_U // NS // ZROWS):
            sl = pl.ds(accbase + t * ZROWS, ZROWS)
            pltpu.sync_copy(zbuf.at[pl.ds(0, ZROWS)], zacc.at[sl])
        ybase = sid * (YROWS // NS)
        pltpu.sync_copy(ycbuf.at[pl.ds(0, YROWS // NS)],
                        ycacc.at[pl.ds(ybase, YROWS // NS)])
        plsc.subcore_barrier()

        row0 = cid * (N // NC) + sid * RPT

        def body(t, _):
            base = row0 + t * CHUNK
            pltpu.sync_copy(ids_hbm.at[pl.ds(base, CHUNK)], idbuf)
            pltpu.sync_copy(z_hbm.at[pl.ds(base, CHUNK)], zbuf)
            pltpu.sync_copy(y_hbm.at[pl.ds(base, CHUNK)], ybuf)
            for g in range(CHUNK // LANES):
                s = pl.ds(g * LANES, LANES)
                idv = idbuf[s]
                yv = ybuf[s]
                idx2buf[s] = lax.shift_right_logical(idv, 3)
                for j in range(LANES):
                    mj = jnp.bitwise_and(idv[j], 7)
                    vec = e0 + yv[j] * e1
                    r = g * LANES + j
                    for q in range(D // LANES):
                        ycbuf[r, pl.ds(q * LANES, LANES)] = jnp.where(
                            mj == q, vec, zerov)
            pltpu.sync_copy(zbuf, zacc.at[idbuf], add=True)
            pltpu.sync_copy(ycbuf, ycacc.at[idx2buf], add=True)
            return 0

        lax.fori_loop(0, RPT // CHUNK, body, 0)
        plsc.subcore_barrier()

        @pl.when(sid == 0)
        def _():
            pltpu.sync_copy(zacc, zout_hbm.at[cid])
            pltpu.sync_copy(ycacc, ycout_hbm.at[cid])
            for t in range(YROWS // CHUNK):
                pltpu.sync_copy(ycacc.at[pl.ds(t * CHUNK, CHUNK)], ycbuf)

                def grp(g, _):
                    cv = zerov
                    for j in range(LANES):
                        rr = g * 2 + j // 8
                        lane = (j % 8) * LANES
                        cj = ycbuf[rr, pl.ds(lane, LANES)][0]
                        cv = jnp.where(iota == j,
                                       jnp.full((LANES,), cj, _f32), cv)
                    cntbuf[pl.ds(g * LANES, LANES)] = cv
                    return 0

                lax.fori_loop(0, CHUNK // 2, grp, 0)
                pltpu.sync_copy(
                    cntbuf,
                    cnt_hbm.at[cid, pl.ds(t * CHUNK * 8, CHUNK * 8)])

    return k(z, ids, y)


def _stage_b(cnt0, cnt1):
    @functools.partial(
        pl.kernel,
        out_type=jax.ShapeDtypeStruct((8, 128), _i32),
        mesh=_mesh(),
        scratch_types=[
            pltpu.VMEM((PAD_U,), _f32),
            pltpu.VMEM((PAD_U,), _f32),
            pltpu.VMEM((8, 128), _i32),
        ],
    )
    def k(cnt0_hbm, cnt1_hbm, offs_hbm, c0buf, c1buf, obuf):
        cid = lax.axis_index("c")
        sid = lax.axis_index("s")

        @pl.when((cid == 0) & (sid == 0))
        def _():
            iota = lax.iota(_i32, LANES)
            pltpu.sync_copy(cnt0_hbm, c0buf)
            pltpu.sync_copy(cnt1_hbm, c1buf)

            pb = []
            for w in range(NW):
                def grp(g, acc, w=w):
                    s = pl.ds(w * CBC + g * LANES, LANES)
                    cnt = c0buf[s] + c1buf[s]
                    t = jnp.int32(0)
                    for j in range(LANES):
                        t = t + (cnt[j] > 0.5).astype(_i32)
                    return acc + t

                pb.append(lax.fori_loop(0, CBC // LANES, grp, 0))

            offp, offa = [], []
            accp = jnp.int32(0)
            acca = jnp.int32(0)
            for w in range(NW):
                offp.append(accp)
                offa.append(acca)
                accp = accp + pb[w]
                acca = acca + (CBC - pb[w])
            u_total = accp
            offa = [u_total + a for a in offa]

            for c in range(NC):
                vp = jnp.zeros((LANES,), _i32)
                va = jnp.zeros((LANES,), _i32)
                for s in range(NS):
                    w = s * NC + c
                    vp = jnp.where(iota == s,
                                   jnp.full((LANES,), offp[w], _i32), vp)
                    va = jnp.where(iota == s,
                                   jnp.full((LANES,), offa[w], _i32), va)
                obuf[c, pl.ds(0, LANES)] = vp
                obuf[2 + c, pl.ds(0, LANES)] = va
            pltpu.sync_copy(obuf, offs_hbm)

    return k(cnt0, cnt1)


def _stage_c(z0, z1, yc0, yc1, cnt0, cnt1, offs):
    @functools.partial(
        pl.kernel,
        out_type=(
            jax.ShapeDtypeStruct((PAD_U, D), _f32),
            jax.ShapeDtypeStruct((PAD_U, D), _f32),
        ),
        mesh=_mesh(),
        scratch_types=[
            pltpu.VMEM((HCB, D), _f32),
            pltpu.VMEM((HCB, D), _f32),
            pltpu.VMEM((HCB // 8 + 4, D), _f32),
            pltpu.VMEM((HCB // 8 + 4, D), _f32),
            pltpu.VMEM((HCB,), _f32),
            pltpu.VMEM((HCB,), _f32),
            pltpu.VMEM((8, 128), _i32),
            pltpu.VMEM((HCB // LANES, LANES), _i32),
            pltpu.SemaphoreType.DMA,
            pltpu.SemaphoreType.DMA,
        ],
    )
    def k(z0_hbm, z1_hbm, yc0_hbm, yc1_hbm, cnt0_hbm, cnt1_hbm, offs_hbm,
          outz_hbm, outyt_hbm, z0c, z1c, yc0c, yc1c, c0v, c1v, obuf,
          idx2d, semz, semy):
        cid = lax.axis_index("c")
        sid = lax.axis_index("s")
        wid = sid * NC + cid
        iota = lax.iota(_i32, LANES)
        vbase = wid * CBC

        pltpu.sync_copy(offs_hbm, obuf)
        rowp = obuf[cid, pl.ds(0, LANES)]
        rowa = obuf[2 + cid, pl.ds(0, LANES)]
        cp0 = jnp.int32(0)
        ca0 = jnp.int32(0)
        for s in range(NS):
            cp0 = jnp.where(sid == s, rowp[s], cp0)
            ca0 = jnp.where(sid == s, rowa[s], ca0)
        carry = (cp0, ca0)

        for h in range(CBC // HCB):
            sl = pl.ds(vbase + h * HCB, HCB)
            ydelta = (h * HCB // 8) % 8
            yoff = pl.multiple_of(vbase // 8 + h * (HCB // 8) - ydelta, 8)
            sly = pl.ds(yoff, HCB // 8 + 4)
            pltpu.sync_copy(z0_hbm.at[sl], z0c)
            pltpu.sync_copy(z1_hbm.at[sl], z1c)
            pltpu.sync_copy(yc0_hbm.at[sly], yc0c)
            pltpu.sync_copy(yc1_hbm.at[sly], yc1c)
            pltpu.sync_copy(cnt0_hbm.at[sl], c0v)
            pltpu.sync_copy(cnt1_hbm.at[sl], c1v)

            def grp(g, carry):
                cp, ca = carry
                cntv = (c0v[pl.ds(g * LANES, LANES)]
                        + c1v[pl.ds(g * LANES, LANES)])
                posv = jnp.zeros((LANES,), _i32)
                for j in range(LANES):
                    pres = cntv[j] > 0.5
                    pj = jnp.where(pres, cp, ca)
                    posv = jnp.where(iota == j, jnp.full((LANES,), pj, _i32),
                                     posv)
                    pi = pres.astype(_i32)
                    cp = cp + pi
                    ca = ca + 1 - pi
                    r = g * LANES + j
                    rr = ydelta + g * 2 + j // 8
                    lane = (j % 8) * LANES
                    yrow = (yc0c[rr, pl.ds(lane, LANES)]
                            + yc1c[rr, pl.ds(lane, LANES)])
                    oy = jnp.where(pres & (yrow[1] >= 0.5 * cntv[j]),
                                   1.0, 0.0)
                    cv = jnp.full((LANES,), cntv[j], _f32)
                    for q in range(D // LANES):
                        s = pl.ds(q * LANES, LANES)
                        z0c[r, s] = (z0c[r, s] + z1c[r, s]) / cv
                    zv = jnp.zeros((LANES,), _f32)
                    z1c[r, pl.ds(0, LANES)] = jnp.where(
                        iota == 0, jnp.full((LANES,), oy, _f32), zv)
                    for q in range(1, D // LANES):
                        z1c[r, pl.ds(q * LANES, LANES)] = zv
                idx2d[g] = posv
                return (cp, ca)

            carry = lax.fori_loop(0, HCB // LANES, grp, carry)

            descs = []
            for b in range(HCB // LANES):
                src = pl.ds(b * LANES, LANES)
                descs.append(pltpu.async_copy(
                    z0c.at[src], outz_hbm.at[idx2d.at[b]], semz))
                descs.append(pltpu.async_copy(
                    z1c.at[src], outyt_hbm.at[idx2d.at[b]], semy))
            for d in descs:
                d.wait()

    return k(z0, z1, yc0, yc1, cnt0, cnt1, offs)


def kernel(z, user_ids, y):
    z = z.astype(_f32)
    y = y.astype(_f32)
    ids = user_ids.astype(_i32)
    zout, ycout, cnt = _stage_a(z, ids, y)
    offs = _stage_b(cnt[0], cnt[1])
    outz, outyt = _stage_c(zout[0], zout[1], ycout[0], ycout[1],
                           cnt[0], cnt[1], offs)
    return outz[:NU], outyt[:NU, 0]
